# baseline (speedup 1.0000x reference)
import jax
import jax.numpy as jnp
from jax import lax
from jax.experimental import pallas as pl
from jax.experimental.pallas import tpu as pltpu, tpu_sc as plsc

B = 16384
D = 64
NC, NS, L = 2, 16, 16
NW = NC * NS
RPW = B // NW
SUB = 128
NSUB = RPW // SUB
NBG = B // SUB
TROW = 8 * SUB


def _body(h_hbm, r_hbm, t_hbm, n_hbm, ent_hbm, rel_hbm, score_hbm, neg_hbm,
          hs, rs, ts, ns,
          hi2, ri2, ti2, ni2,
          hb0, rb0, tb0, nb0, hb1, rb1, tb1, nb1,
          so0, no0, so1, no1,
          isem, gsem0, gsem1, osem0, osem1):
    wid = lax.axis_index("s") * NC + lax.axis_index("c")
    base_w = wid * RPW

    cps = [pltpu.async_copy(h_hbm.at[pl.ds(base_w, RPW)], hs, isem),
           pltpu.async_copy(r_hbm.at[pl.ds(base_w, RPW)], rs, isem),
           pltpu.async_copy(t_hbm.at[pl.ds(base_w, RPW)], ts, isem),
           pltpu.async_copy(n_hbm.at[pl.ds(base_w, RPW)], ns, isem)]
    for c in cps:
        c.wait()

    for src, idx2 in ((hs, hi2), (rs, ri2), (ts, ti2), (ns, ni2)):
        for v in range(RPW // L):
            sub, lo = (v * L) // SUB, (v * L) % SUB
            idx2[sub, pl.ds(lo, L)] = src[pl.ds(v * L, L)]

    gsets = ((hb0, rb0, tb0, nb0, gsem0),
             (hb1, rb1, tb1, nb1, gsem1))
    osets = ((so0, no0, osem0), (so1, no1, osem1))

    def start_gathers(k):
        hb, rb, tb, nb, gsem = gsets[k % 2]
        return [pltpu.async_copy(ent_hbm.at[hi2.at[k]], hb, gsem),
                pltpu.async_copy(rel_hbm.at[ri2.at[k]], rb, gsem),
                pltpu.async_copy(ent_hbm.at[ti2.at[k]], tb, gsem),
                pltpu.async_copy(ent_hbm.at[ni2.at[k]], nb, gsem)]

    iota = lax.iota(jnp.int32, L)
    cflat = [((iota + j * L) // 8) * TROW + ((iota + j * L) % 8) * SUB
             for j in range(D // L)]
    pend_g = {0: start_gathers(0)}
    pend_o = {}
    for k in range(NSUB):
        hb, rb, tb, nb, _ = gsets[k % 2]
        outS, outN, osem = osets[k % 2]
        if k + 1 < NSUB:
            pend_g[k + 1] = start_gathers(k + 1)
        for c in pend_o.pop(k - 2, ()):
            c.wait()
        for c in pend_g.pop(k):
            c.wait()

        @plsc.parallel_loop(0, SUB)
        def _compute(i):
            co = i + jnp.zeros((L,), jnp.int32)
            for j in range(D // L):
                sl = pl.ds(j * L, L)
                s = hb[i, sl] + rb[i, sl]
                idx = cflat[j] + co
                plsc.store_scatter(outS, [idx], s - tb[i, sl])
                plsc.store_scatter(outN, [idx], s - nb[i, sl])

        bg = wid * NSUB + k
        pend_o[k] = [
            c
            for dg in range(D // 8)
            for c in (
                pltpu.async_copy(
                    outS.at[pl.ds(dg * TROW, TROW)],
                    score_hbm.at[pl.ds(dg * (NBG * TROW) + bg * TROW, TROW)],
                    osem),
                pltpu.async_copy(
                    outN.at[pl.ds(dg * TROW, TROW)],
                    neg_hbm.at[pl.ds(dg * (NBG * TROW) + bg * TROW, TROW)],
                    osem),
            )
        ]
    for b in sorted(pend_o):
        for c in pend_o[b]:
            c.wait()


def kernel(h, r, t, neg_idx, entity_table, relation_table):
    mesh = plsc.VectorSubcoreMesh(
        core_axis_name="c", subcore_axis_name="s",
        num_cores=NC, num_subcores=NS)
    f = pl.kernel(
        _body,
        out_type=(jax.ShapeDtypeStruct((D * B,), jnp.float32),
                  jax.ShapeDtypeStruct((D * B,), jnp.float32)),
        mesh=mesh,
        compiler_params=pltpu.CompilerParams(
            needs_layout_passes=False, use_tc_tiling_on_sc=False),
        scratch_types=[
            pltpu.VMEM((RPW,), jnp.int32),
            pltpu.VMEM((RPW,), jnp.int32),
            pltpu.VMEM((RPW,), jnp.int32),
            pltpu.VMEM((RPW,), jnp.int32),
            pltpu.VMEM((NSUB, SUB), jnp.int32),
            pltpu.VMEM((NSUB, SUB), jnp.int32),
            pltpu.VMEM((NSUB, SUB), jnp.int32),
            pltpu.VMEM((NSUB, SUB), jnp.int32),
            pltpu.VMEM((SUB, D), jnp.float32),
            pltpu.VMEM((SUB, D), jnp.float32),
            pltpu.VMEM((SUB, D), jnp.float32),
            pltpu.VMEM((SUB, D), jnp.float32),
            pltpu.VMEM((SUB, D), jnp.float32),
            pltpu.VMEM((SUB, D), jnp.float32),
            pltpu.VMEM((SUB, D), jnp.float32),
            pltpu.VMEM((SUB, D), jnp.float32),
            pltpu.VMEM((D * SUB,), jnp.float32),
            pltpu.VMEM((D * SUB,), jnp.float32),
            pltpu.VMEM((D * SUB,), jnp.float32),
            pltpu.VMEM((D * SUB,), jnp.float32),
            pltpu.SemaphoreType.DMA,
            pltpu.SemaphoreType.DMA,
            pltpu.SemaphoreType.DMA,
            pltpu.SemaphoreType.DMA,
            pltpu.SemaphoreType.DMA,
        ],
    )
    score4, neg4 = f(h.astype(jnp.int32), r.astype(jnp.int32),
                     t.astype(jnp.int32), neg_idx.astype(jnp.int32),
                     entity_table, relation_table)

    def unpack(o):
        o = o.reshape(D // 8, NBG, 8, SUB).transpose(0, 2, 1, 3)
        return o.reshape(D, B).T[:, None, :]

    return unpack(score4), unpack(neg4)


# speedup vs baseline: 1.0119x; 1.0119x over previous
import jax
import jax.numpy as jnp
from jax import lax
from jax.experimental import pallas as pl
from jax.experimental.pallas import tpu as pltpu, tpu_sc as plsc

B = 16384
D = 64
NC, NS, L = 2, 16, 16
NW = NC * NS
RPW = B // NW
SUB = 128
NSUB = RPW // SUB
NBG = B // SUB
TD = D // 8
TROW = 8 * SUB


def _body(h_hbm, r_hbm, t_hbm, n_hbm, ent_hbm, rel_hbm, score_hbm, neg_hbm,
          hs, rs, ts, ns,
          hi2, ri2, ti2, ni2,
          hb0, rb0, tb0, nb0, hb1, rb1, tb1, nb1,
          so0, no0, so1, no1,
          isem, gsem0, gsem1, osem0, osem1):
    wid = lax.axis_index("s") * NC + lax.axis_index("c")
    base_w = wid * RPW

    cps = [pltpu.async_copy(h_hbm.at[pl.ds(base_w, RPW)], hs, isem),
           pltpu.async_copy(r_hbm.at[pl.ds(base_w, RPW)], rs, isem),
           pltpu.async_copy(t_hbm.at[pl.ds(base_w, RPW)], ts, isem),
           pltpu.async_copy(n_hbm.at[pl.ds(base_w, RPW)], ns, isem)]
    for c in cps:
        c.wait()

    for src, idx2 in ((hs, hi2), (rs, ri2), (ts, ti2), (ns, ni2)):
        for v in range(RPW // L):
            sub, lo = (v * L) // SUB, (v * L) % SUB
            idx2[sub, pl.ds(lo, L)] = src[pl.ds(v * L, L)]

    gsets = ((hb0, rb0, tb0, nb0, gsem0),
             (hb1, rb1, tb1, nb1, gsem1))
    osets = ((so0, no0, osem0), (so1, no1, osem1))

    def start_gathers(k):
        hb, rb, tb, nb, gsem = gsets[k % 2]
        return [pltpu.async_copy(ent_hbm.at[hi2.at[k]], hb, gsem),
                pltpu.async_copy(rel_hbm.at[ri2.at[k]], rb, gsem),
                pltpu.async_copy(ent_hbm.at[ti2.at[k]], tb, gsem),
                pltpu.async_copy(ent_hbm.at[ni2.at[k]], nb, gsem)]

    iota = lax.iota(jnp.int32, L)
    cjhi = [(iota + j * L) // 8 for j in range(D // L)]
    cjlo = [((iota + j * L) % 8) * SUB for j in range(D // L)]
    pend_g = {0: start_gathers(0)}
    pend_o = {}
    for k in range(NSUB):
        hb, rb, tb, nb, _ = gsets[k % 2]
        outS, outN, osem = osets[k % 2]
        if k + 1 < NSUB:
            pend_g[k + 1] = start_gathers(k + 1)
        for c in pend_o.pop(k - 2, ()):
            c.wait()
        for c in pend_g.pop(k):
            c.wait()

        @plsc.parallel_loop(0, SUB)
        def _compute(i):
            co = i + jnp.zeros((L,), jnp.int32)
            for j in range(D // L):
                sl = pl.ds(j * L, L)
                s = hb[i, sl] + rb[i, sl]
                col = cjlo[j] + co
                plsc.store_scatter(outS, [cjhi[j], col], s - tb[i, sl])
                plsc.store_scatter(outN, [cjhi[j], col], s - nb[i, sl])

        bg = wid * NSUB + k
        pend_o[k] = [
            pltpu.async_copy(outS, score_hbm.at[:, bg], osem),
            pltpu.async_copy(outN, neg_hbm.at[:, bg], osem)]
    for b in sorted(pend_o):
        for c in pend_o[b]:
            c.wait()


def kernel(h, r, t, neg_idx, entity_table, relation_table):
    mesh = plsc.VectorSubcoreMesh(
        core_axis_name="c", subcore_axis_name="s",
        num_cores=NC, num_subcores=NS)
    f = pl.kernel(
        _body,
        out_type=(jax.ShapeDtypeStruct((TD, NBG, TROW), jnp.float32),
                  jax.ShapeDtypeStruct((TD, NBG, TROW), jnp.float32)),
        mesh=mesh,
        compiler_params=pltpu.CompilerParams(
            needs_layout_passes=False, use_tc_tiling_on_sc=False),
        scratch_types=[
            pltpu.VMEM((RPW,), jnp.int32),
            pltpu.VMEM((RPW,), jnp.int32),
            pltpu.VMEM((RPW,), jnp.int32),
            pltpu.VMEM((RPW,), jnp.int32),
            pltpu.VMEM((NSUB, SUB), jnp.int32),
            pltpu.VMEM((NSUB, SUB), jnp.int32),
            pltpu.VMEM((NSUB, SUB), jnp.int32),
            pltpu.VMEM((NSUB, SUB), jnp.int32),
            pltpu.VMEM((SUB, D), jnp.float32),
            pltpu.VMEM((SUB, D), jnp.float32),
            pltpu.VMEM((SUB, D), jnp.float32),
            pltpu.VMEM((SUB, D), jnp.float32),
            pltpu.VMEM((SUB, D), jnp.float32),
            pltpu.VMEM((SUB, D), jnp.float32),
            pltpu.VMEM((SUB, D), jnp.float32),
            pltpu.VMEM((SUB, D), jnp.float32),
            pltpu.VMEM((TD, TROW), jnp.float32),
            pltpu.VMEM((TD, TROW), jnp.float32),
            pltpu.VMEM((TD, TROW), jnp.float32),
            pltpu.VMEM((TD, TROW), jnp.float32),
            pltpu.SemaphoreType.DMA,
            pltpu.SemaphoreType.DMA,
            pltpu.SemaphoreType.DMA,
            pltpu.SemaphoreType.DMA,
            pltpu.SemaphoreType.DMA,
        ],
    )
    score4, neg4 = f(h.astype(jnp.int32), r.astype(jnp.int32),
                     t.astype(jnp.int32), neg_idx.astype(jnp.int32),
                     entity_table, relation_table)

    def unpack(o):
        o = o.reshape(TD, NBG, 8, SUB).transpose(0, 2, 1, 3)
        return o.reshape(D, B).T[:, None, :]

    return unpack(score4), unpack(neg4)


# speedup vs baseline: 1.0121x; 1.0002x over previous
import jax
import jax.numpy as jnp
from jax import lax
from jax.experimental import pallas as pl
from jax.experimental.pallas import tpu as pltpu, tpu_sc as plsc

B = 16384
D = 64
NC, NS, L = 2, 16, 16
NW = NC * NS
RPW = B // NW
SUB = 128
NSUB = RPW // SUB
NBG = B // SUB
TD = D // 8
TROW = 8 * SUB


def _body(h_hbm, r_hbm, t_hbm, n_hbm, ent_hbm, rel_hbm, score_hbm, neg_hbm,
          hs, rs, ts, ns,
          hb0, rb0, tb0, nb0, hb1, rb1, tb1, nb1,
          so0, no0, so1, no1,
          isem, gsem0, gsem1, osem0, osem1):
    wid = lax.axis_index("s") * NC + lax.axis_index("c")
    base_w = wid * RPW

    cps = [pltpu.async_copy(h_hbm.at[pl.ds(base_w, RPW)], hs, isem),
           pltpu.async_copy(r_hbm.at[pl.ds(base_w, RPW)], rs, isem),
           pltpu.async_copy(t_hbm.at[pl.ds(base_w, RPW)], ts, isem),
           pltpu.async_copy(n_hbm.at[pl.ds(base_w, RPW)], ns, isem)]
    for c in cps:
        c.wait()

    gsets = ((hb0, rb0, tb0, nb0, gsem0),
             (hb1, rb1, tb1, nb1, gsem1))
    osets = ((so0, no0, osem0), (so1, no1, osem1))

    def start_gathers(k):
        hb, rb, tb, nb, gsem = gsets[k % 2]
        sl = pl.ds(k * SUB, SUB)
        return [pltpu.async_copy(ent_hbm.at[hs.at[sl]], hb, gsem),
                pltpu.async_copy(rel_hbm.at[rs.at[sl]], rb, gsem),
                pltpu.async_copy(ent_hbm.at[ts.at[sl]], tb, gsem),
                pltpu.async_copy(ent_hbm.at[ns.at[sl]], nb, gsem)]

    iota = lax.iota(jnp.int32, L)
    cjhi = [(iota + j * L) // 8 for j in range(D // L)]
    cjlo = [((iota + j * L) % 8) * SUB for j in range(D // L)]
    pend_g = {0: start_gathers(0)}
    pend_o = {}
    for k in range(NSUB):
        hb, rb, tb, nb, _ = gsets[k % 2]
        outS, outN, osem = osets[k % 2]
        if k + 1 < NSUB:
            pend_g[k + 1] = start_gathers(k + 1)
        for c in pend_o.pop(k - 2, ()):
            c.wait()
        for c in pend_g.pop(k):
            c.wait()

        @plsc.parallel_loop(0, SUB)
        def _compute(i):
            co = i + jnp.zeros((L,), jnp.int32)
            for j in range(D // L):
                sl = pl.ds(j * L, L)
                s = hb[i, sl] + rb[i, sl]
                col = cjlo[j] + co
                plsc.store_scatter(outS, [cjhi[j], col], s - tb[i, sl])
                plsc.store_scatter(outN, [cjhi[j], col], s - nb[i, sl])

        bg = wid * NSUB + k
        pend_o[k] = [
            pltpu.async_copy(outS, score_hbm.at[:, bg], osem),
            pltpu.async_copy(outN, neg_hbm.at[:, bg], osem)]
    for b in sorted(pend_o):
        for c in pend_o[b]:
            c.wait()


def kernel(h, r, t, neg_idx, entity_table, relation_table):
    mesh = plsc.VectorSubcoreMesh(
        core_axis_name="c", subcore_axis_name="s",
        num_cores=NC, num_subcores=NS)
    f = pl.kernel(
        _body,
        out_type=(jax.ShapeDtypeStruct((TD, NBG, TROW), jnp.float32),
                  jax.ShapeDtypeStruct((TD, NBG, TROW), jnp.float32)),
        mesh=mesh,
        compiler_params=pltpu.CompilerParams(
            needs_layout_passes=False, use_tc_tiling_on_sc=False),
        scratch_types=[
            pltpu.VMEM((RPW,), jnp.int32),
            pltpu.VMEM((RPW,), jnp.int32),
            pltpu.VMEM((RPW,), jnp.int32),
            pltpu.VMEM((RPW,), jnp.int32),
            pltpu.VMEM((SUB, D), jnp.float32),
            pltpu.VMEM((SUB, D), jnp.float32),
            pltpu.VMEM((SUB, D), jnp.float32),
            pltpu.VMEM((SUB, D), jnp.float32),
            pltpu.VMEM((SUB, D), jnp.float32),
            pltpu.VMEM((SUB, D), jnp.float32),
            pltpu.VMEM((SUB, D), jnp.float32),
            pltpu.VMEM((SUB, D), jnp.float32),
            pltpu.VMEM((TD, TROW), jnp.float32),
            pltpu.VMEM((TD, TROW), jnp.float32),
            pltpu.VMEM((TD, TROW), jnp.float32),
            pltpu.VMEM((TD, TROW), jnp.float32),
            pltpu.SemaphoreType.DMA,
            pltpu.SemaphoreType.DMA,
            pltpu.SemaphoreType.DMA,
            pltpu.SemaphoreType.DMA,
            pltpu.SemaphoreType.DMA,
        ],
    )
    score4, neg4 = f(h.astype(jnp.int32), r.astype(jnp.int32),
                     t.astype(jnp.int32), neg_idx.astype(jnp.int32),
                     entity_table, relation_table)

    def unpack(o):
        o = o.reshape(TD, NBG, 8, SUB).transpose(0, 2, 1, 3)
        return o.reshape(D, B).T[:, None, :]

    return unpack(score4), unpack(neg4)


# speedup vs baseline: 1.0188x; 1.0066x over previous
import jax
import jax.numpy as jnp
from jax import lax
from jax.experimental import pallas as pl
from jax.experimental.pallas import tpu as pltpu, tpu_sc as plsc

B = 16384
D = 64
NC, NS, L = 2, 16, 16
NW = NC * NS
RPW = B // NW
SUB = 64
NSUB = RPW // SUB
BLK = 128
NBG = B // BLK
TD = D // 8


def _body(h_hbm, r_hbm, t_hbm, n_hbm, ent_hbm, rel_hbm, score_hbm, neg_hbm,
          hs, rs, ts, ns,
          hi2, ri2, ti2, ni2,
          hb0, rb0, tb0, nb0, hb1, rb1, tb1, nb1,
          so0, no0, so1, no1,
          isem, gsem0, gsem1, osem0, osem1):
    wid = lax.axis_index("s") * NC + lax.axis_index("c")
    base_w = wid * RPW

    cps = [pltpu.async_copy(h_hbm.at[pl.ds(base_w, RPW)], hs, isem),
           pltpu.async_copy(r_hbm.at[pl.ds(base_w, RPW)], rs, isem),
           pltpu.async_copy(t_hbm.at[pl.ds(base_w, RPW)], ts, isem),
           pltpu.async_copy(n_hbm.at[pl.ds(base_w, RPW)], ns, isem)]
    for c in cps:
        c.wait()

    for src, idx2 in ((hs, hi2), (rs, ri2), (ts, ti2), (ns, ni2)):
        for v in range(RPW // L):
            sub, lo = (v * L) // SUB, (v * L) % SUB
            idx2[sub, pl.ds(lo, L)] = src[pl.ds(v * L, L)]

    gsets = ((hb0, rb0, tb0, nb0, gsem0),
             (hb1, rb1, tb1, nb1, gsem1))
    osets = ((so0, no0, osem0), (so1, no1, osem1))

    def start_gathers(k):
        hb, rb, tb, nb, gsem = gsets[k % 2]
        return [pltpu.async_copy(ent_hbm.at[hi2.at[k]], hb, gsem),
                pltpu.async_copy(rel_hbm.at[ri2.at[k]], rb, gsem),
                pltpu.async_copy(ent_hbm.at[ti2.at[k]], tb, gsem),
                pltpu.async_copy(ent_hbm.at[ni2.at[k]], nb, gsem)]

    iota = lax.iota(jnp.int32, L)
    cjhi = [(iota + j * L) // 8 for j in range(D // L)]
    cjlo = [((iota + j * L) % 8) * BLK for j in range(D // L)]
    pend_g = {0: start_gathers(0)}
    pend_o = {}
    for k in range(NSUB):
        hb, rb, tb, nb, _ = gsets[k % 2]
        blk = k // 2
        outS, outN, osem = osets[blk % 2]
        if k + 1 < NSUB:
            pend_g[k + 1] = start_gathers(k + 1)
        if k % 2 == 0:
            for c in pend_o.pop(blk - 2, ()):
                c.wait()
        for c in pend_g.pop(k):
            c.wait()

        half = (k % 2) * SUB

        @plsc.parallel_loop(0, SUB)
        def _compute(i):
            co = (i + half) + jnp.zeros((L,), jnp.int32)
            for j in range(D // L):
                sl = pl.ds(j * L, L)
                s = hb[i, sl] + rb[i, sl]
                col = cjlo[j] + co
                plsc.store_scatter(outS, [cjhi[j], col], s - tb[i, sl])
                plsc.store_scatter(outN, [cjhi[j], col], s - nb[i, sl])

        if k % 2 == 1:
            bg = (base_w + blk * BLK) // BLK
            pend_o[blk] = [
                pltpu.async_copy(outS, score_hbm.at[:, bg], osem),
                pltpu.async_copy(outN, neg_hbm.at[:, bg], osem)]
    for b in sorted(pend_o):
        for c in pend_o[b]:
            c.wait()


def kernel(h, r, t, neg_idx, entity_table, relation_table):
    mesh = plsc.VectorSubcoreMesh(
        core_axis_name="c", subcore_axis_name="s",
        num_cores=NC, num_subcores=NS)
    f = pl.kernel(
        _body,
        out_type=(jax.ShapeDtypeStruct((TD, NBG, 8 * BLK), jnp.float32),
                  jax.ShapeDtypeStruct((TD, NBG, 8 * BLK), jnp.float32)),
        mesh=mesh,
        compiler_params=pltpu.CompilerParams(
            needs_layout_passes=False, use_tc_tiling_on_sc=False),
        scratch_types=[
            pltpu.VMEM((RPW,), jnp.int32),
            pltpu.VMEM((RPW,), jnp.int32),
            pltpu.VMEM((RPW,), jnp.int32),
            pltpu.VMEM((RPW,), jnp.int32),
            pltpu.VMEM((NSUB, SUB), jnp.int32),
            pltpu.VMEM((NSUB, SUB), jnp.int32),
            pltpu.VMEM((NSUB, SUB), jnp.int32),
            pltpu.VMEM((NSUB, SUB), jnp.int32),
            pltpu.VMEM((SUB, D), jnp.float32),
            pltpu.VMEM((SUB, D), jnp.float32),
            pltpu.VMEM((SUB, D), jnp.float32),
            pltpu.VMEM((SUB, D), jnp.float32),
            pltpu.VMEM((SUB, D), jnp.float32),
            pltpu.VMEM((SUB, D), jnp.float32),
            pltpu.VMEM((SUB, D), jnp.float32),
            pltpu.VMEM((SUB, D), jnp.float32),
            pltpu.VMEM((TD, 8 * BLK), jnp.float32),
            pltpu.VMEM((TD, 8 * BLK), jnp.float32),
            pltpu.VMEM((TD, 8 * BLK), jnp.float32),
            pltpu.VMEM((TD, 8 * BLK), jnp.float32),
            pltpu.SemaphoreType.DMA,
            pltpu.SemaphoreType.DMA,
            pltpu.SemaphoreType.DMA,
            pltpu.SemaphoreType.DMA,
            pltpu.SemaphoreType.DMA,
        ],
    )
    score4, neg4 = f(h.astype(jnp.int32), r.astype(jnp.int32),
                     t.astype(jnp.int32), neg_idx.astype(jnp.int32),
                     entity_table, relation_table)

    def unpack(o):
        o = o.reshape(TD, NBG, 8, BLK).transpose(0, 2, 1, 3)
        return o.reshape(D, B).T[:, None, :]

    return unpack(score4), unpack(neg4)
